# baseline (speedup 1.0000x reference)
import jax
import jax.numpy as jnp
from jax import lax
from jax.experimental import pallas as pl
from jax.experimental.pallas import tpu as pltpu
from jax.experimental.pallas import tpu_sc as plsc

B = 4
N = 400000
CUBE = 128
ELEMS_PER_B = CUBE * CUBE * CUBE * 4
NFLOAT = B * ELEMS_PER_B
NTILES = 32
P2 = N // 8
GP = 64
CH = 2048
NCH_FULL = P2 // CH
TAIL = P2 - NCH_FULL * CH
TAIL_FULL_GROUPS = TAIL // GP
TAIL_REM_VECS = (TAIL - TAIL_FULL_GROUPS * GP) // 16
ZCH = 8192
NZCH = NFLOAT // NTILES // ZCH
MAGIC = float(1.5 * 2 ** 23)
PLANE = B * N
LFULL = ((CH + 127) // 128) * 512 + 512
LTAIL = ((TAIL + 127) // 128) * 512 + 512
LSEC = LFULL


def _sc_body(pts0, pts1, cons, vals, out,
             ptsbuf, consbuf, valbuf, idxbuf0, idxbuf1,
             zbuf, sem0, sem1, zsem):
    c = lax.axis_index("c")
    s = lax.axis_index("s")
    b = c * 2 + s // 8
    sub = s % 8

    zero16 = jnp.zeros((16,), jnp.float32)

    def zfill(j, _):
        zbuf[pl.ds(j * 16, 16)] = zero16
        return 0

    lax.fori_loop(0, ZCH // 16, zfill, 0)
    zoff = (c * 16 + s) * (NFLOAT // NTILES)

    def zbody(k, _):
        @pl.when(k >= 4)
        def _():
            pltpu.make_async_copy(
                zbuf, out.at[pl.ds(pl.multiple_of(zoff + (k - 4) * ZCH, 8), ZCH)],
                zsem).wait()
        pltpu.async_copy(
            zbuf, out.at[pl.ds(pl.multiple_of(zoff + k * ZCH, 8), ZCH)], zsem)
        return 0

    lax.fori_loop(0, NZCH, zbody, 0)
    for k in range(NZCH - 4, NZCH):
        pltpu.make_async_copy(
            zbuf, out.at[pl.ds(pl.multiple_of(zoff + k * ZCH, 8), ZCH)],
            zsem).wait()
    plsc.subcore_barrier()

    pltpu.sync_copy(cons.at[pl.ds(pl.multiple_of(b * 96, 8), 96)], consbuf)
    hx = consbuf[pl.ds(0, 16)]
    hy = consbuf[pl.ds(16, 16)]
    hz = consbuf[pl.ds(32, 16)]
    ivx = consbuf[pl.ds(48, 16)]
    ivy = consbuf[pl.ds(64, 16)]
    ivz = consbuf[pl.ds(80, 16)]
    iota = lax.iota(jnp.int32, 16)
    n_base = sub * P2

    for t, pts in ((0, pts0), (1, pts1)):
        for q in range(8):
            pltpu.sync_copy(vals.at[pl.ds(t * 128, 128)],
                            valbuf.at[pl.ds(q * 128, 128)])
        ebase = b * ELEMS_PER_B + t * (2 * CUBE)

        def emit_vec(ibuf, q, ib0, v, rmax):
            idxv = iota + ib0(v)
            x = plsc.load_gather(ptsbuf, [idxv])
            y = plsc.load_gather(ptsbuf, [idxv + LSEC])
            z = plsc.load_gather(ptsbuf, [idxv + 2 * LSEC])
            fx = ((x - hx) * ivx + MAGIC) - MAGIC
            fy = ((y - hy) * ivy + MAGIC) - MAGIC
            fz = ((z - hz) * ivz + MAGIC) - MAGIC
            xi = fx.astype(jnp.int32) + (CUBE // 2)
            yi = fy.astype(jnp.int32) + (CUBE // 2)
            zi = fz.astype(jnp.int32) + (CUBE // 2)
            valid = ((xi >= 0) & (xi < CUBE)
                     & (yi >= 0) & (yi < CUBE)
                     & (zi >= 0) & (zi < CUBE))
            ea = (xi * (CUBE * 4 * CUBE) + yi * (4 * CUBE) + zi + ebase)
            ea = jnp.where(valid, ea, -1)
            rmax = jnp.maximum(rmax, jnp.max(ea))
            ibuf[pl.ds(q * 128 + v * 16, 16)] = ea
            ibuf[pl.ds(q * 128 + 64 + v * 16, 16)] = ea + CUBE
            return rmax

        def emit_super(ibuf, sem, stage, rel0, nvecs, rmax, started):
            @pl.when(started)
            def _():
                pltpu.make_async_copy(valbuf, out.at[ibuf], sem).wait()

            rm = rmax
            for q in range(8):
                for v in range(nvecs[q]):
                    rm = emit_vec(ibuf, q,
                                  ib0_maker(stage, rel0 + q * GP), v, rm)
            rmv = jnp.broadcast_to(rm, (16,))
            for q in range(8):
                for v in range(4):
                    if v < nvecs[q]:
                        lo = ibuf[pl.ds(q * 128 + v * 16, 16)]
                        ok = lo >= 0
                        ibuf[pl.ds(q * 128 + v * 16, 16)] = (
                            jnp.where(ok, lo, rmv))
                        hi = ibuf[pl.ds(q * 128 + 64 + v * 16, 16)]
                        ibuf[pl.ds(q * 128 + 64 + v * 16, 16)] = (
                            jnp.where(ok, hi, rmv + CUBE))
                    else:
                        ibuf[pl.ds(q * 128 + v * 16, 16)] = rmv
                        ibuf[pl.ds(q * 128 + 64 + v * 16, 16)] = rmv + CUBE

            @pl.when(rm >= 0)
            def _():
                pltpu.async_copy(valbuf, out.at[ibuf], sem)

            return rm, started | (rm >= 0)

        FULLV = (4,) * 8
        TAILV = (4, 4, 4, 4, 4, 1, 0, 0)

        def stage_chunk(n0, npts, L):
            cbs = (n0 // 128) * 512
            w0 = cbs + b * 128
            ws = jnp.minimum(w0, PLANE - L)
            delta = w0 - ws
            for k in range(3):
                pltpu.sync_copy(
                    pts.at[pl.ds(pl.multiple_of(k * PLANE + ws, 8), L)],
                    ptsbuf.at[pl.ds(k * LSEC, L)])
            return delta, n0 % 128

        def ib0_maker(stage, rel):
            delta, rem0 = stage

            def ib0(v):
                q = rem0 + rel + v * 16
                return delta + (q // 128) * 512 + q % 128
            return ib0

        def chunk_body(j, carry):
            rmax, st0, st1 = carry
            n0 = n_base + j * CH
            nb0 = stage_chunk(n0, CH, LFULL)

            def pair_body(k, carry):
                rmax, st0, st1 = carry
                rmax, st0 = emit_super(idxbuf0, sem0, nb0, (2 * k) * 512,
                                       FULLV, rmax, st0)
                rmax, st1 = emit_super(idxbuf1, sem1, nb0, (2 * k + 1) * 512,
                                       FULLV, rmax, st1)
                return rmax, st0, st1

            return lax.fori_loop(0, CH // 1024, pair_body, (rmax, st0, st1))

        carry = lax.fori_loop(0, NCH_FULL, chunk_body,
                              (jnp.int32(-1), jnp.bool_(False),
                               jnp.bool_(False)))
        rmax, st0, st1 = carry

        tn0 = n_base + NCH_FULL * CH
        tnb0 = stage_chunk(tn0, TAIL, LTAIL)
        rmax, st0 = emit_super(idxbuf0, sem0, tnb0, 0, FULLV, rmax, st0)
        rmax, st1 = emit_super(idxbuf1, sem1, tnb0, 512, TAILV, rmax, st1)

        @pl.when(st0)
        def _():
            pltpu.make_async_copy(valbuf, out.at[idxbuf0], sem0).wait()

        @pl.when(st1)
        def _():
            pltpu.make_async_copy(valbuf, out.at[idxbuf1], sem1).wait()


def _make_sc_kernel():
    mesh = plsc.VectorSubcoreMesh(core_axis_name="c", subcore_axis_name="s",
                                  num_cores=2, num_subcores=16)
    return pl.kernel(
        _sc_body,
        out_type=jax.ShapeDtypeStruct((NFLOAT,), jnp.float32),
        mesh=mesh,
        compiler_params=pltpu.CompilerParams(needs_layout_passes=False),
        scratch_types=[
            pltpu.VMEM((3 * LSEC,), jnp.float32),
            pltpu.VMEM((96,), jnp.float32),
            pltpu.VMEM((1024,), jnp.float32),
            pltpu.VMEM((1024,), jnp.int32),
            pltpu.VMEM((1024,), jnp.int32),
            pltpu.VMEM((ZCH,), jnp.float32),
            pltpu.SemaphoreType.DMA,
            pltpu.SemaphoreType.DMA,
            pltpu.SemaphoreType.DMA,
        ],
    )


_sc_kernel = _make_sc_kernel()


def kernel(map_points_lane, map_points_crosswalk, neck_voxel_sizes, emb_weight):
    def native_flat(p):
        return (p.transpose(2, 1, 0).reshape(3, N // 128, 128, B)
                .swapaxes(2, 3).reshape(-1))

    pts0 = native_flat(map_points_lane)
    pts1 = native_flat(map_points_crosswalk)
    c6 = jnp.concatenate([neck_voxel_sizes * 0.5, 1.0 / neck_voxel_sizes],
                         axis=1)
    cons = jnp.broadcast_to(c6[:, :, None], (B, 6, 16)).reshape(-1)
    cons = jnp.asarray(cons, jnp.float32)
    vals = jnp.broadcast_to(emb_weight[:, :, None], (2, 2, 64)).reshape(-1)
    vals = jnp.asarray(vals, jnp.float32)
    flat = _sc_kernel(pts0, pts1, cons, vals)
    return flat.reshape(B, CUBE, CUBE, 4, CUBE).swapaxes(3, 4)


# speedup vs baseline: 1.0068x; 1.0068x over previous
import jax
import jax.numpy as jnp
from jax import lax
from jax.experimental import pallas as pl
from jax.experimental.pallas import tpu as pltpu
from jax.experimental.pallas import tpu_sc as plsc

B = 4
N = 400000
CUBE = 128
ELEMS_PER_B = CUBE * CUBE * CUBE * 4
NFLOAT = B * ELEMS_PER_B
NTILES = 32
P2 = N // 8
GP = 64
CH = 2048
NCH_FULL = P2 // CH
TAIL = P2 - NCH_FULL * CH
TAIL_FULL_GROUPS = TAIL // GP
TAIL_REM_VECS = (TAIL - TAIL_FULL_GROUPS * GP) // 16
ZCH = 8192
NZCH = NFLOAT // NTILES // ZCH
MAGIC = float(1.5 * 2 ** 23)
PLANE = B * N
LFULL = ((CH + 127) // 128) * 512 + 512
LTAIL = ((TAIL + 127) // 128) * 512 + 512
LSEC = LFULL


def _sc_body(pts0, pts1, cons, vals, out,
             ptsbuf, consbuf, valbuf, idxbuf0, idxbuf1, idxbuf2, idxbuf3,
             zbuf, sem0, sem1, sem2, sem3, zsem):
    c = lax.axis_index("c")
    s = lax.axis_index("s")
    b = c * 2 + s // 8
    sub = s % 8

    zero16 = jnp.zeros((16,), jnp.float32)

    def zfill(j, _):
        zbuf[pl.ds(j * 16, 16)] = zero16
        return 0

    lax.fori_loop(0, ZCH // 16, zfill, 0)
    zoff = (c * 16 + s) * (NFLOAT // NTILES)

    def zbody(k, _):
        @pl.when(k >= 4)
        def _():
            pltpu.make_async_copy(
                zbuf, out.at[pl.ds(pl.multiple_of(zoff + (k - 4) * ZCH, 8), ZCH)],
                zsem).wait()
        pltpu.async_copy(
            zbuf, out.at[pl.ds(pl.multiple_of(zoff + k * ZCH, 8), ZCH)], zsem)
        return 0

    lax.fori_loop(0, NZCH, zbody, 0)
    for k in range(NZCH - 4, NZCH):
        pltpu.make_async_copy(
            zbuf, out.at[pl.ds(pl.multiple_of(zoff + k * ZCH, 8), ZCH)],
            zsem).wait()
    plsc.subcore_barrier()

    pltpu.sync_copy(cons.at[pl.ds(pl.multiple_of(b * 96, 8), 96)], consbuf)
    hx = consbuf[pl.ds(0, 16)]
    hy = consbuf[pl.ds(16, 16)]
    hz = consbuf[pl.ds(32, 16)]
    ivx = consbuf[pl.ds(48, 16)]
    ivy = consbuf[pl.ds(64, 16)]
    ivz = consbuf[pl.ds(80, 16)]
    iota = lax.iota(jnp.int32, 16)
    n_base = sub * P2

    for t, pts in ((0, pts0), (1, pts1)):
        pltpu.sync_copy(vals.at[pl.ds(t * 128, 128)], valbuf)
        ebase = b * ELEMS_PER_B + t * (2 * CUBE)

        def emit_vec(ibuf, ib0, v, rmax):
            idxv = iota + ib0(v)
            x = plsc.load_gather(ptsbuf, [idxv])
            y = plsc.load_gather(ptsbuf, [idxv + LSEC])
            z = plsc.load_gather(ptsbuf, [idxv + 2 * LSEC])
            fx = ((x - hx) * ivx + MAGIC) - MAGIC
            fy = ((y - hy) * ivy + MAGIC) - MAGIC
            fz = ((z - hz) * ivz + MAGIC) - MAGIC
            xi = fx.astype(jnp.int32) + (CUBE // 2)
            yi = fy.astype(jnp.int32) + (CUBE // 2)
            zi = fz.astype(jnp.int32) + (CUBE // 2)
            valid = ((xi >= 0) & (xi < CUBE)
                     & (yi >= 0) & (yi < CUBE)
                     & (zi >= 0) & (zi < CUBE))
            ea = (xi * (CUBE * 4 * CUBE) + yi * (4 * CUBE) + zi + ebase)
            ea = jnp.where(valid, ea, -1)
            rmax = jnp.maximum(rmax, jnp.max(ea))
            ibuf[pl.ds(v * 16, 16)] = ea
            ibuf[pl.ds(64 + v * 16, 16)] = ea + CUBE
            return rmax

        def emit_group(ibuf, sem, g0, nvec, rmax, started):
            @pl.when(started)
            def _():
                pltpu.make_async_copy(valbuf, out.at[ibuf], sem).wait()

            rm = rmax
            for v in range(nvec):
                rm = emit_vec(ibuf, g0, v, rm)
            rmv = jnp.broadcast_to(rm, (16,))
            for v in range(4):
                if v < nvec:
                    lo = ibuf[pl.ds(v * 16, 16)]
                    ok = lo >= 0
                    ibuf[pl.ds(v * 16, 16)] = jnp.where(ok, lo, rmv)
                    hi = ibuf[pl.ds(64 + v * 16, 16)]
                    ibuf[pl.ds(64 + v * 16, 16)] = jnp.where(ok, hi, rmv + CUBE)
                else:
                    ibuf[pl.ds(v * 16, 16)] = rmv
                    ibuf[pl.ds(64 + v * 16, 16)] = rmv + CUBE

            @pl.when(rm >= 0)
            def _():
                pltpu.async_copy(valbuf, out.at[ibuf], sem)

            return rm, started | (rm >= 0)

        def stage_chunk(n0, npts, L):
            cbs = (n0 // 128) * 512
            w0 = cbs + b * 128
            ws = jnp.minimum(w0, PLANE - L)
            delta = w0 - ws
            for k in range(3):
                pltpu.sync_copy(
                    pts.at[pl.ds(pl.multiple_of(k * PLANE + ws, 8), L)],
                    ptsbuf.at[pl.ds(k * LSEC, L)])
            return delta, n0 % 128

        def ib0_maker(stage, rel):
            delta, rem0 = stage

            def ib0(v):
                q = rem0 + rel + v * 16
                return delta + (q // 128) * 512 + q % 128
            return ib0

        def chunk_body(j, carry):
            rmax, st = carry
            n0 = n_base + j * CH
            nb0 = stage_chunk(n0, CH, LFULL)

            def quad_body(k, carry):
                rmax, st = carry
                sts = []
                for q, (ib, sm) in enumerate(((idxbuf0, sem0), (idxbuf1, sem1),
                                              (idxbuf2, sem2), (idxbuf3, sem3))):
                    rmax, stq = emit_group(ib, sm,
                                           ib0_maker(nb0, (4 * k + q) * GP),
                                           GP // 16, rmax, st[q])
                    sts.append(stq)
                return rmax, tuple(sts)

            return lax.fori_loop(0, CH // GP // 4, quad_body, (rmax, st))

        carry = lax.fori_loop(0, NCH_FULL, chunk_body,
                              (jnp.int32(-1), (jnp.bool_(False),) * 4))
        rmax, st = carry

        tn0 = n_base + NCH_FULL * CH
        tnb0 = stage_chunk(tn0, TAIL, LTAIL)

        def tail_body(g, carry):
            rmax, st0 = carry
            return emit_group(idxbuf0, sem0, ib0_maker(tnb0, g * GP),
                              GP // 16, rmax, st0)

        st0 = st[0]
        rmax, st0 = lax.fori_loop(0, TAIL_FULL_GROUPS, tail_body, (rmax, st0))
        rmax, st0 = emit_group(idxbuf0, sem0,
                               ib0_maker(tnb0, TAIL_FULL_GROUPS * GP),
                               TAIL_REM_VECS, rmax, st0)
        st = (st0,) + st[1:]

        for stq, ib, sm in ((st[0], idxbuf0, sem0), (st[1], idxbuf1, sem1),
                            (st[2], idxbuf2, sem2), (st[3], idxbuf3, sem3)):
            @pl.when(stq)
            def _(ib=ib, sm=sm):
                pltpu.make_async_copy(valbuf, out.at[ib], sm).wait()


def _make_sc_kernel():
    mesh = plsc.VectorSubcoreMesh(core_axis_name="c", subcore_axis_name="s",
                                  num_cores=2, num_subcores=16)
    return pl.kernel(
        _sc_body,
        out_type=jax.ShapeDtypeStruct((NFLOAT,), jnp.float32),
        mesh=mesh,
        compiler_params=pltpu.CompilerParams(needs_layout_passes=False),
        scratch_types=[
            pltpu.VMEM((3 * LSEC,), jnp.float32),
            pltpu.VMEM((96,), jnp.float32),
            pltpu.VMEM((128,), jnp.float32),
            pltpu.VMEM((128,), jnp.int32),
            pltpu.VMEM((128,), jnp.int32),
            pltpu.VMEM((128,), jnp.int32),
            pltpu.VMEM((128,), jnp.int32),
            pltpu.VMEM((ZCH,), jnp.float32),
            pltpu.SemaphoreType.DMA,
            pltpu.SemaphoreType.DMA,
            pltpu.SemaphoreType.DMA,
            pltpu.SemaphoreType.DMA,
            pltpu.SemaphoreType.DMA,
        ],
    )


_sc_kernel = _make_sc_kernel()


def kernel(map_points_lane, map_points_crosswalk, neck_voxel_sizes, emb_weight):
    def native_flat(p):
        return (p.transpose(2, 1, 0).reshape(3, N // 128, 128, B)
                .swapaxes(2, 3).reshape(-1))

    pts0 = native_flat(map_points_lane)
    pts1 = native_flat(map_points_crosswalk)
    c6 = jnp.concatenate([neck_voxel_sizes * 0.5, 1.0 / neck_voxel_sizes],
                         axis=1)
    cons = jnp.broadcast_to(c6[:, :, None], (B, 6, 16)).reshape(-1)
    cons = jnp.asarray(cons, jnp.float32)
    vals = jnp.broadcast_to(emb_weight[:, :, None], (2, 2, 64)).reshape(-1)
    vals = jnp.asarray(vals, jnp.float32)
    flat = _sc_kernel(pts0, pts1, cons, vals)
    return flat.reshape(B, CUBE, CUBE, 4, CUBE).swapaxes(3, 4)
